# baseline (speedup 1.0000x reference)
import functools

import jax
import jax.numpy as jnp
from jax.experimental import pallas as pl
from jax.experimental.pallas import tpu as pltpu
from jax.experimental.pallas import tpu_sc as plsc

_ROTS = (13, 15, 26, 6, 17, 29, 16, 24)
_KS = (0, 1, 0x1BD11BDB)

_LANES = 128
_NEG_INF = float("-inf")


def _rotl(x, r):
    return (x << jnp.uint32(r)) | (x >> jnp.uint32(32 - r))


def _threefry_bits(flat_idx_u32):
    u32 = lambda v: jnp.uint32(v)
    x1 = flat_idx_u32 + u32(_KS[1])
    x0 = x1
    x1 = _rotl(x1, _ROTS[0]) ^ x0
    for r in _ROTS[1:4]:
        x0 = x0 + x1
        x1 = _rotl(x1, r) ^ x0
    x0 = x0 + u32(_KS[1])
    x1 = x1 + u32(_KS[2] + 1)
    for i in range(1, 5):
        for r in (_ROTS[4:] if i % 2 == 1 else _ROTS[:4]):
            x0 = x0 + x1
            x1 = _rotl(x1, r) ^ x0
        x0 = x0 + u32(_KS[(i + 1) % 3])
        x1 = x1 + u32(_KS[(i + 2) % 3] + (i + 1))
    return x0 ^ x1


def _gumbel(flat_idx_u32):
    bits = _threefry_bits(flat_idx_u32)
    fbits = (bits >> jnp.uint32(9)) | jnp.uint32(0x3F800000)
    floats = jax.lax.bitcast_convert_type(fbits, jnp.float32) - jnp.float32(1.0)
    minv = jnp.float32(1e-9)
    u = jnp.maximum(minv, floats + minv)
    return -jnp.log(-jnp.log(u))


_SLAB = 512


def _argmax_kernel(base_ref, logits_ref, idx_ref, zero_ref, vmax_ref,
                   vcol_ref, *, rows, bv, vocab, nblk):
    nb = pl.program_id(1)
    zero_ref[...] = jnp.zeros((rows, bv), jnp.float32)

    @pl.when(nb == 0)
    def _init():
        vmax_ref[...] = jnp.full((rows, _SLAB), _NEG_INF, jnp.float32)
        vcol_ref[...] = jnp.zeros((rows, _SLAB), jnp.int32)

    base = nb * bv
    row0 = pl.program_id(0) * rows + base_ref[0]
    row_mul = (jax.lax.broadcasted_iota(jnp.int32, (rows, _SLAB), 0)
               + row0) * vocab
    lane = jax.lax.broadcasted_iota(jnp.int32, (rows, _SLAB), 1)

    rv = vmax_ref[...]
    rc = vcol_ref[...]
    for s in range(bv // _SLAB):
        cols = lane + (base + s * _SLAB)
        flat = (row_mul + cols).astype(jnp.uint32)
        z = logits_ref[:, s * _SLAB:(s + 1) * _SLAB] + _gumbel(flat)
        z = jnp.where(cols < vocab, z, _NEG_INF)
        better = z > rv
        rv = jnp.where(better, z, rv)
        rc = jnp.where(better, cols, rc)
    vmax_ref[...] = rv
    vcol_ref[...] = rc

    @pl.when(nb == nblk - 1)
    def _finish():
        rowmax = jnp.max(rv, axis=1, keepdims=True)
        ccand = jnp.where(rv == rowmax, rc, jnp.int32(2**31 - 1))
        best = jnp.min(ccand, axis=1, keepdims=True)
        idx_ref[...] = jnp.broadcast_to(best, (rows, _LANES))


_ZW = 100000


def _sc_zeros(batch, vocab):
    zw = _ZW if vocab % _ZW == 0 else vocab
    nwin = vocab // zw

    @pl.kernel(
        out_type=jax.ShapeDtypeStruct((batch, vocab), jnp.float32),
        mesh=plsc.VectorSubcoreMesh(core_axis_name="core",
                                    subcore_axis_name="subcore"),
        scratch_types=[pltpu.VMEM((zw,), jnp.float32),
                       pltpu.SemaphoreType.DMA],
    )
    def zero_kernel(o_hbm, zbuf, sem):
        row = jax.lax.axis_index("core") * 16 + jax.lax.axis_index("subcore")

        @pl.when(row < batch)
        def _guarded():
            @pl.loop(0, zw, step=16)
            def _fill(c):
                zbuf[pl.ds(c, 16)] = jnp.zeros((16,), jnp.float32)

            @pl.loop(0, nwin)
            def _stream(w):
                pltpu.async_copy(zbuf, o_hbm.at[row, pl.ds(w * zw, zw)],
                                 sem).wait()

    return zero_kernel()


_PW = 512


def _patch_kernel(idx_ref, zin_ref, out_ref):
    del zin_ref
    i = pl.program_id(0)
    a = (i // 8) * 8
    base = (idx_ref[i] // _PW) * _PW
    sub = jax.lax.broadcasted_iota(jnp.int32, (8, _PW), 0)
    lane = jax.lax.broadcasted_iota(jnp.int32, (8, _PW), 1)
    tvec = jnp.zeros((8, _PW), jnp.int32)
    for s in range(8):
        tvec = jnp.where(sub == s, idx_ref[a + s], tvec)
    out_ref[...] = jnp.where(tvec - base == lane, jnp.float32(1.0),
                             jnp.float32(0.0))


def kernel(logits):
    devs = jax.devices()
    if len(devs) >= 2 and logits.shape[0] % 2 == 0:
        mesh = jax.sharding.Mesh(devs[:2], ("x",))
        spec = jax.sharding.PartitionSpec("x", None)
        from jax.experimental.shard_map import shard_map
        half = logits.shape[0] // 2

        def _sharded(x):
            row_base = jax.lax.axis_index("x").astype(jnp.int32) * half
            return _kernel_local(x, row_base)

        return shard_map(_sharded, mesh=mesh, in_specs=(spec,),
                         out_specs=spec, check_rep=False)(logits)
    return _kernel_local(logits, jnp.int32(0))


def _kernel_local(logits, row_base):
    batch, vocab = logits.shape
    groups = 2
    rows = batch // groups
    bv = 16384
    nblk = pl.cdiv(vocab, bv)

    idx, zeros = pl.pallas_call(
        functools.partial(_argmax_kernel, rows=rows, bv=bv, vocab=vocab,
                          nblk=nblk),
        grid=(groups, nblk),
        in_specs=[pl.BlockSpec(memory_space=pltpu.SMEM),
                  pl.BlockSpec((rows, bv), lambda g, nb: (g, nb))],
        out_specs=[pl.BlockSpec((rows, _LANES), lambda g, nb: (g, 0)),
                   pl.BlockSpec((rows, bv), lambda g, nb: (g, nb))],
        out_shape=[jax.ShapeDtypeStruct((batch, _LANES), jnp.int32),
                   jax.ShapeDtypeStruct((batch, vocab), jnp.float32)],
        scratch_shapes=[
            pltpu.VMEM((rows, _SLAB), jnp.float32),
            pltpu.VMEM((rows, _SLAB), jnp.int32),
        ],
        compiler_params=pltpu.CompilerParams(
            dimension_semantics=("parallel", "arbitrary")),
    )(jnp.reshape(row_base, (1,)), logits)

    grid_spec = pltpu.PrefetchScalarGridSpec(
        num_scalar_prefetch=1,
        grid=(batch,),
        in_specs=[pl.BlockSpec(memory_space=pl.ANY)],
        out_specs=pl.BlockSpec(
            (8, _PW),
            lambda i, idx_ref: (i // 8, idx_ref[i] // _PW)),
    )
    out = pl.pallas_call(
        _patch_kernel,
        grid_spec=grid_spec,
        out_shape=jax.ShapeDtypeStruct((batch, vocab), jnp.float32),
        input_output_aliases={1: 0},
        compiler_params=pltpu.CompilerParams(
            dimension_semantics=("arbitrary",)),
    )(idx[:, 0], zeros)
    return out


# speedup vs baseline: 1.3923x; 1.3923x over previous
import functools

import jax
import jax.numpy as jnp
from jax.experimental import pallas as pl
from jax.experimental.pallas import tpu as pltpu
from jax.experimental.pallas import tpu_sc as plsc

_ROTS = (13, 15, 26, 6, 17, 29, 16, 24)
_KS = (0, 1, 0x1BD11BDB)

_LANES = 128
_NEG_INF = float("-inf")


def _rotl(x, r):
    return (x << jnp.uint32(r)) | (x >> jnp.uint32(32 - r))


def _threefry_bits(flat_idx_u32):
    u32 = lambda v: jnp.uint32(v)
    x1 = flat_idx_u32 + u32(_KS[1])
    x0 = x1
    x1 = _rotl(x1, _ROTS[0]) ^ x0
    for r in _ROTS[1:4]:
        x0 = x0 + x1
        x1 = _rotl(x1, r) ^ x0
    x0 = x0 + u32(_KS[1])
    x1 = x1 + u32(_KS[2] + 1)
    for i in range(1, 5):
        for r in (_ROTS[4:] if i % 2 == 1 else _ROTS[:4]):
            x0 = x0 + x1
            x1 = _rotl(x1, r) ^ x0
        x0 = x0 + u32(_KS[(i + 1) % 3])
        x1 = x1 + u32(_KS[(i + 2) % 3] + (i + 1))
    return x0 ^ x1


def _gumbel(flat_idx_u32):
    bits = _threefry_bits(flat_idx_u32)
    fbits = (bits >> jnp.uint32(9)) | jnp.uint32(0x3F800000)
    floats = jax.lax.bitcast_convert_type(fbits, jnp.float32) - jnp.float32(1.0)
    minv = jnp.float32(1e-9)
    u = jnp.maximum(minv, floats + minv)
    return -jnp.log(-jnp.log(u))


_SLAB = 512


def _argmax_kernel(base_ref, logits_ref, idx_ref, zero_ref, vmax_ref,
                   vcol_ref, *, rows, bv, vocab, nblk):
    nb = pl.program_id(1)
    zero_ref[...] = jnp.zeros((rows, bv), jnp.float32)

    @pl.when(nb == 0)
    def _init():
        vmax_ref[...] = jnp.full((rows, _SLAB), _NEG_INF, jnp.float32)
        vcol_ref[...] = jnp.zeros((rows, _SLAB), jnp.int32)

    base = nb * bv
    row0 = pl.program_id(0) * rows + base_ref[0]
    row_mul = (jax.lax.broadcasted_iota(jnp.int32, (rows, _SLAB), 0)
               + row0) * vocab
    lane = jax.lax.broadcasted_iota(jnp.int32, (rows, _SLAB), 1)

    rv = vmax_ref[...]
    rc = vcol_ref[...]
    for s in range(bv // _SLAB):
        cols = lane + (base + s * _SLAB)
        flat = (row_mul + cols).astype(jnp.uint32)
        z = logits_ref[:, s * _SLAB:(s + 1) * _SLAB] + _gumbel(flat)
        z = jnp.where(cols < vocab, z, _NEG_INF)
        better = z > rv
        rv = jnp.where(better, z, rv)
        rc = jnp.where(better, cols, rc)
    vmax_ref[...] = rv
    vcol_ref[...] = rc

    @pl.when(nb == nblk - 1)
    def _finish():
        rowmax = jnp.max(rv, axis=1, keepdims=True)
        ccand = jnp.where(rv == rowmax, rc, jnp.int32(2**31 - 1))
        best = jnp.min(ccand, axis=1, keepdims=True)
        idx_ref[...] = jnp.broadcast_to(best, (rows, _LANES))


_ZW = 100000


def _sc_zeros(batch, vocab):
    zw = _ZW if vocab % _ZW == 0 else vocab
    nwin = vocab // zw

    @pl.kernel(
        out_type=jax.ShapeDtypeStruct((batch, vocab), jnp.float32),
        mesh=plsc.VectorSubcoreMesh(core_axis_name="core",
                                    subcore_axis_name="subcore"),
        scratch_types=[pltpu.VMEM((zw,), jnp.float32),
                       pltpu.SemaphoreType.DMA],
    )
    def zero_kernel(o_hbm, zbuf, sem):
        row = jax.lax.axis_index("core") * 16 + jax.lax.axis_index("subcore")

        @pl.when(row < batch)
        def _guarded():
            @pl.loop(0, zw, step=16)
            def _fill(c):
                zbuf[pl.ds(c, 16)] = jnp.zeros((16,), jnp.float32)

            @pl.loop(0, nwin)
            def _stream(w):
                pltpu.async_copy(zbuf, o_hbm.at[row, pl.ds(w * zw, zw)],
                                 sem).wait()

    return zero_kernel()


_PW = 512


def _patch_kernel(idx_ref, zin_ref, out_ref):
    del zin_ref
    i = pl.program_id(0)
    a = (i // 8) * 8
    base = (idx_ref[i] // _PW) * _PW
    sub = jax.lax.broadcasted_iota(jnp.int32, (8, _PW), 0)
    lane = jax.lax.broadcasted_iota(jnp.int32, (8, _PW), 1)
    tvec = jnp.zeros((8, _PW), jnp.int32)
    for s in range(8):
        tvec = jnp.where(sub == s, idx_ref[a + s], tvec)
    out_ref[...] = jnp.where(tvec - base == lane, jnp.float32(1.0),
                             jnp.float32(0.0))


def kernel(logits):
    return _kernel_local(logits, jnp.int32(0))


def _kernel_local(logits, row_base):
    batch, vocab = logits.shape
    groups = 2
    rows = batch // groups
    bv = 32768
    nblk = pl.cdiv(vocab, bv)

    idx, zeros = pl.pallas_call(
        functools.partial(_argmax_kernel, rows=rows, bv=bv, vocab=vocab,
                          nblk=nblk),
        grid=(groups, nblk),
        in_specs=[pl.BlockSpec(memory_space=pltpu.SMEM),
                  pl.BlockSpec((rows, bv), lambda g, nb: (g, nb))],
        out_specs=[pl.BlockSpec((rows, _LANES), lambda g, nb: (g, 0)),
                   pl.BlockSpec((rows, bv), lambda g, nb: (g, nb))],
        out_shape=[jax.ShapeDtypeStruct((batch, _LANES), jnp.int32),
                   jax.ShapeDtypeStruct((batch, vocab), jnp.float32)],
        scratch_shapes=[
            pltpu.VMEM((rows, _SLAB), jnp.float32),
            pltpu.VMEM((rows, _SLAB), jnp.int32),
        ],
        compiler_params=pltpu.CompilerParams(
            dimension_semantics=("parallel", "arbitrary")),
    )(jnp.reshape(row_base, (1,)), logits)

    grid_spec = pltpu.PrefetchScalarGridSpec(
        num_scalar_prefetch=1,
        grid=(batch,),
        in_specs=[pl.BlockSpec(memory_space=pl.ANY)],
        out_specs=pl.BlockSpec(
            (8, _PW),
            lambda i, idx_ref: (i // 8, idx_ref[i] // _PW)),
    )
    out = pl.pallas_call(
        _patch_kernel,
        grid_spec=grid_spec,
        out_shape=jax.ShapeDtypeStruct((batch, vocab), jnp.float32),
        input_output_aliases={1: 0},
        compiler_params=pltpu.CompilerParams(
            dimension_semantics=("arbitrary",)),
    )(idx[:, 0], zeros)
    return out


# speedup vs baseline: 1.4222x; 1.0215x over previous
import functools

import jax
import jax.numpy as jnp
from jax.experimental import pallas as pl
from jax.experimental.pallas import tpu as pltpu
from jax.experimental.pallas import tpu_sc as plsc

_ROTS = (13, 15, 26, 6, 17, 29, 16, 24)
_KS = (0, 1, 0x1BD11BDB)

_LANES = 128
_NEG_INF = float("-inf")


def _rotl(x, r):
    return (x << jnp.uint32(r)) | (x >> jnp.uint32(32 - r))


def _threefry_bits(flat_idx_u32):
    u32 = lambda v: jnp.uint32(v)
    x1 = flat_idx_u32 + u32(_KS[1])
    x0 = x1
    x1 = _rotl(x1, _ROTS[0]) ^ x0
    for r in _ROTS[1:4]:
        x0 = x0 + x1
        x1 = _rotl(x1, r) ^ x0
    x0 = x0 + u32(_KS[1])
    x1 = x1 + u32(_KS[2] + 1)
    for i in range(1, 5):
        for r in (_ROTS[4:] if i % 2 == 1 else _ROTS[:4]):
            x0 = x0 + x1
            x1 = _rotl(x1, r) ^ x0
        x0 = x0 + u32(_KS[(i + 1) % 3])
        x1 = x1 + u32(_KS[(i + 2) % 3] + (i + 1))
    return x0 ^ x1


def _gumbel(flat_idx_u32):
    bits = _threefry_bits(flat_idx_u32)
    fbits = (bits >> jnp.uint32(9)) | jnp.uint32(0x3F800000)
    floats = jax.lax.bitcast_convert_type(fbits, jnp.float32) - jnp.float32(1.0)
    minv = jnp.float32(1e-9)
    u = jnp.maximum(minv, floats + minv)
    return -jnp.log(-jnp.log(u))


_SLAB = 512


def _argmax_kernel(base_ref, logits_ref, idx_ref, zero_ref, vmax_ref,
                   vcol_ref, *, rows, bv, vocab, nblk):
    nb = pl.program_id(1)
    zero_ref[...] = jnp.zeros((rows, bv), jnp.float32)

    @pl.when(nb == 0)
    def _init():
        vmax_ref[...] = jnp.full((rows, _SLAB), _NEG_INF, jnp.float32)
        vcol_ref[...] = jnp.zeros((rows, _SLAB), jnp.int32)

    base = nb * bv
    row0 = pl.program_id(0) * rows + base_ref[0]
    lane = jax.lax.broadcasted_iota(jnp.int32, (rows, _SLAB), 1)
    rowlane = (jax.lax.broadcasted_iota(jnp.int32, (rows, _SLAB), 0)
               + row0) * vocab + lane

    def _scan(masked):
        rv = vmax_ref[...]
        rc = vcol_ref[...]
        for s in range(bv // _SLAB):
            sc = base + s * _SLAB
            flat = (rowlane + sc).astype(jnp.uint32)
            z = logits_ref[:, s * _SLAB:(s + 1) * _SLAB] + _gumbel(flat)
            if masked:
                z = jnp.where(lane < vocab - sc, z, _NEG_INF)
            better = z > rv
            rv = jnp.where(better, z, rv)
            rc = jnp.where(better, sc, rc)
        vmax_ref[...] = rv
        vcol_ref[...] = rc
        return rv, rc

    @pl.when(nb < nblk - 1)
    def _main():
        _scan(False)

    @pl.when(nb == nblk - 1)
    def _finish():
        rv, rc = _scan(True)
        rcl = rc + lane
        rowmax = jnp.max(rv, axis=1, keepdims=True)
        ccand = jnp.where(rv == rowmax, rcl, jnp.int32(2**31 - 1))
        best = jnp.min(ccand, axis=1, keepdims=True)
        idx_ref[...] = jnp.broadcast_to(best, (rows, _LANES))


_ZW = 100000


def _sc_zeros(batch, vocab):
    zw = _ZW if vocab % _ZW == 0 else vocab
    nwin = vocab // zw

    @pl.kernel(
        out_type=jax.ShapeDtypeStruct((batch, vocab), jnp.float32),
        mesh=plsc.VectorSubcoreMesh(core_axis_name="core",
                                    subcore_axis_name="subcore"),
        scratch_types=[pltpu.VMEM((zw,), jnp.float32),
                       pltpu.SemaphoreType.DMA],
    )
    def zero_kernel(o_hbm, zbuf, sem):
        row = jax.lax.axis_index("core") * 16 + jax.lax.axis_index("subcore")

        @pl.when(row < batch)
        def _guarded():
            @pl.loop(0, zw, step=16)
            def _fill(c):
                zbuf[pl.ds(c, 16)] = jnp.zeros((16,), jnp.float32)

            @pl.loop(0, nwin)
            def _stream(w):
                pltpu.async_copy(zbuf, o_hbm.at[row, pl.ds(w * zw, zw)],
                                 sem).wait()

    return zero_kernel()


_PW = 512


def _patch_kernel(idx_ref, zin_ref, out_ref):
    del zin_ref
    i = pl.program_id(0)
    a = (i // 8) * 8
    base = (idx_ref[i] // _PW) * _PW
    sub = jax.lax.broadcasted_iota(jnp.int32, (8, _PW), 0)
    lane = jax.lax.broadcasted_iota(jnp.int32, (8, _PW), 1)
    tvec = jnp.zeros((8, _PW), jnp.int32)
    for s in range(8):
        tvec = jnp.where(sub == s, idx_ref[a + s], tvec)
    out_ref[...] = jnp.where(tvec - base == lane, jnp.float32(1.0),
                             jnp.float32(0.0))


def kernel(logits):
    return _kernel_local(logits, jnp.int32(0))


def _kernel_local(logits, row_base):
    batch, vocab = logits.shape
    groups = 2
    rows = batch // groups
    bv = 32768
    nblk = pl.cdiv(vocab, bv)

    idx, zeros = pl.pallas_call(
        functools.partial(_argmax_kernel, rows=rows, bv=bv, vocab=vocab,
                          nblk=nblk),
        grid=(groups, nblk),
        in_specs=[pl.BlockSpec(memory_space=pltpu.SMEM),
                  pl.BlockSpec((rows, bv), lambda g, nb: (g, nb))],
        out_specs=[pl.BlockSpec((rows, _LANES), lambda g, nb: (g, 0)),
                   pl.BlockSpec((rows, bv), lambda g, nb: (g, nb))],
        out_shape=[jax.ShapeDtypeStruct((batch, _LANES), jnp.int32),
                   jax.ShapeDtypeStruct((batch, vocab), jnp.float32)],
        scratch_shapes=[
            pltpu.VMEM((rows, _SLAB), jnp.float32),
            pltpu.VMEM((rows, _SLAB), jnp.int32),
        ],
        compiler_params=pltpu.CompilerParams(
            dimension_semantics=("parallel", "arbitrary")),
    )(jnp.reshape(row_base, (1,)), logits)

    grid_spec = pltpu.PrefetchScalarGridSpec(
        num_scalar_prefetch=1,
        grid=(batch,),
        in_specs=[pl.BlockSpec(memory_space=pl.ANY)],
        out_specs=pl.BlockSpec(
            (8, _PW),
            lambda i, idx_ref: (i // 8, idx_ref[i] // _PW)),
    )
    out = pl.pallas_call(
        _patch_kernel,
        grid_spec=grid_spec,
        out_shape=jax.ShapeDtypeStruct((batch, vocab), jnp.float32),
        input_output_aliases={1: 0},
        compiler_params=pltpu.CompilerParams(
            dimension_semantics=("arbitrary",)),
    )(idx[:, 0], zeros)
    return out
